# baseline (speedup 1.0000x reference)
import functools

import jax
import jax.numpy as jnp
from jax import lax
from jax.experimental import pallas as pl
from jax.experimental.pallas import tpu as pltpu
from jax.experimental.pallas import tpu_sc as plsc

VOCAB = 100000
EMB = 64
HID = 128
B = 4096
L = 50

NC = 2
NS = 16
NW = NC * NS
ROWS_PER_W = B // NW
RPC = 2
IDX_PER_CHUNK = RPC * L
NCHUNK = ROWS_PER_W // RPC
NBUF = 4
SEG = EMB // 16


def _pool_body(x_hbm, emb_hbm, out_hbm, idx_v, rows_v, out_v, *sems):
    wid = lax.axis_index("s") * NC + lax.axis_index("c")

    pltpu.sync_copy(x_hbm.at[wid], idx_v)

    def start(j, b):
        pltpu.async_copy(emb_hbm.at[idx_v.at[j]], rows_v.at[b], sems[b])

    def wait(b):
        pltpu.make_async_copy(emb_hbm.at[idx_v.at[0]], rows_v.at[b], sems[b]).wait()

    for b in range(NBUF):
        start(b, b)

    inv_l = jnp.float32(1.0 / L)

    @pl.loop(0, NCHUNK, step=NBUF)
    def _outer(g):
        for b in range(NBUF):
            j = g + b
            wait(b)
            for r in range(RPC):
                def inner(l, accs):
                    base = r * L + l
                    return tuple(
                        accs[d] + rows_v[b, base, pl.ds(d * 16, 16)]
                        for d in range(SEG)
                    )
                accs = lax.fori_loop(
                    0, L, inner,
                    tuple(jnp.zeros((16,), jnp.float32) for _ in range(SEG)),
                )
                row = j * RPC + r
                for d in range(SEG):
                    out_v[row, pl.ds(d * 16, 16)] = accs[d] * inv_l
            nxt = j + NBUF

            @pl.when(nxt < NCHUNK)
            def _():
                start(nxt, b)

    pltpu.sync_copy(out_v, out_hbm.at[pl.ds(wid * ROWS_PER_W, ROWS_PER_W)])


@jax.jit
def _pool(x_grouped, emb):
    mesh = plsc.VectorSubcoreMesh(
        core_axis_name="c", subcore_axis_name="s", num_cores=NC, num_subcores=NS
    )
    f = pl.kernel(
        _pool_body,
        out_type=jax.ShapeDtypeStruct((B, EMB), jnp.float32),
        mesh=mesh,
        scratch_types=[
            pltpu.VMEM((NCHUNK, IDX_PER_CHUNK), jnp.int32),
            pltpu.VMEM((NBUF, IDX_PER_CHUNK, EMB), jnp.float32),
            pltpu.VMEM((ROWS_PER_W, EMB), jnp.float32),
        ] + [pltpu.SemaphoreType.DMA] * NBUF,
        compiler_params=pltpu.CompilerParams(use_tc_tiling_on_sc=False),
    )
    return f(x_grouped, emb)


BV = 1536
NVB = (VOCAB + BV - 1) // BV


def _mlp_body(pooled_ref, w1_ref, b1_ref, w2t_ref, b2_ref, out_ref, h_ref):
    @pl.when(pl.program_id(0) == 0)
    def _():
        h = jnp.dot(pooled_ref[...], w1_ref[...],
                    preferred_element_type=jnp.float32)
        h_ref[...] = jnp.maximum(h + b1_ref[...], 0.0)

    out_ref[...] = lax.dot_general(
        w2t_ref[...], h_ref[...],
        (((1,), (1,)), ((), ())),
        preferred_element_type=jnp.float32,
    ) + lax.dot_general(
        b2_ref[...], jnp.ones((1, B), jnp.float32),
        (((0,), (0,)), ((), ())),
        preferred_element_type=jnp.float32,
    )


@jax.jit
def _mlp(pooled, w1, b1, w2t, b2r):
    return pl.pallas_call(
        _mlp_body,
        grid=(NVB,),
        in_specs=[
            pl.BlockSpec((B, EMB), lambda j: (0, 0)),
            pl.BlockSpec((EMB, HID), lambda j: (0, 0)),
            pl.BlockSpec((1, HID), lambda j: (0, 0)),
            pl.BlockSpec((BV, HID), lambda j: (j, 0)),
            pl.BlockSpec((1, BV), lambda j: (0, j)),
        ],
        out_specs=pl.BlockSpec((BV, B), lambda j: (j, 0)),
        out_shape=jax.ShapeDtypeStruct((VOCAB, B), jnp.float32),
        scratch_shapes=[pltpu.VMEM((B, HID), jnp.float32)],
        compiler_params=pltpu.CompilerParams(
            dimension_semantics=("arbitrary",),
            vmem_limit_bytes=60 * 1024 * 1024,
        ),
    )(pooled, w1, b1, w2t, b2r)


def kernel(x, emb, W1, b1, W2, b2):
    x_grouped = x.reshape(NW, NCHUNK, IDX_PER_CHUNK).astype(jnp.int32)
    pooled = _pool(x_grouped, emb)
    logits_t = _mlp(pooled, W1, b1.reshape(1, HID), W2.T,
                    b2.reshape(1, VOCAB))
    return logits_t.T


# speedup vs baseline: 1.0003x; 1.0003x over previous
import functools

import jax
import jax.numpy as jnp
from jax import lax
from jax.experimental import pallas as pl
from jax.experimental.pallas import tpu as pltpu
from jax.experimental.pallas import tpu_sc as plsc

VOCAB = 100000
EMB = 64
HID = 128
B = 4096
L = 50

NC = 2
NS = 16
NW = NC * NS
ROWS_PER_W = B // NW
RPC = 2
IDX_PER_CHUNK = RPC * L
NCHUNK = ROWS_PER_W // RPC
NBUF = 4
SEG = EMB // 16


def _pool_body(x_hbm, emb_hbm, out_hbm, idx_v, rows_v, out_v, *sems):
    wid = lax.axis_index("s") * NC + lax.axis_index("c")

    pltpu.sync_copy(x_hbm.at[wid], idx_v)

    def start(j, b):
        pltpu.async_copy(emb_hbm.at[idx_v.at[j]], rows_v.at[b], sems[b])

    def wait(b):
        pltpu.make_async_copy(emb_hbm.at[idx_v.at[0]], rows_v.at[b], sems[b]).wait()

    for b in range(NBUF):
        start(b, b)

    inv_l = jnp.float32(1.0 / L)

    @pl.loop(0, NCHUNK, step=NBUF)
    def _outer(g):
        for b in range(NBUF):
            j = g + b
            wait(b)
            for r in range(RPC):
                def inner(l, accs):
                    base = r * L + l
                    return tuple(
                        accs[d] + rows_v[b, base, pl.ds(d * 16, 16)]
                        for d in range(SEG)
                    )
                accs = lax.fori_loop(
                    0, L, inner,
                    tuple(jnp.zeros((16,), jnp.float32) for _ in range(SEG)),
                )
                row = j * RPC + r
                for d in range(SEG):
                    out_v[row, pl.ds(d * 16, 16)] = accs[d] * inv_l
            nxt = j + NBUF

            @pl.when(nxt < NCHUNK)
            def _():
                start(nxt, b)

    pltpu.sync_copy(out_v, out_hbm.at[pl.ds(wid * ROWS_PER_W, ROWS_PER_W)])


@jax.jit
def _pool(x_grouped, emb):
    mesh = plsc.VectorSubcoreMesh(
        core_axis_name="c", subcore_axis_name="s", num_cores=NC, num_subcores=NS
    )
    f = pl.kernel(
        _pool_body,
        out_type=jax.ShapeDtypeStruct((B, EMB), jnp.float32),
        mesh=mesh,
        scratch_types=[
            pltpu.VMEM((NCHUNK, IDX_PER_CHUNK), jnp.int32),
            pltpu.VMEM((NBUF, IDX_PER_CHUNK, EMB), jnp.float32),
            pltpu.VMEM((ROWS_PER_W, EMB), jnp.float32),
        ] + [pltpu.SemaphoreType.DMA] * NBUF,
        compiler_params=pltpu.CompilerParams(use_tc_tiling_on_sc=False),
    )
    return f(x_grouped, emb)


BV = 1024
NVB = (VOCAB + BV - 1) // BV


def _mlp_body(pooled_ref, w1_ref, b1_ref, w2t_ref, b2_ref, out_ref, h_ref):
    @pl.when(pl.program_id(0) == 0)
    def _():
        h = jnp.dot(pooled_ref[...], w1_ref[...],
                    preferred_element_type=jnp.float32)
        h_ref[...] = jnp.maximum(h + b1_ref[...], 0.0)

    out_ref[...] = lax.dot_general(
        w2t_ref[...], h_ref[...],
        (((1,), (1,)), ((), ())),
        preferred_element_type=jnp.float32,
    ) + lax.dot_general(
        b2_ref[...], jnp.ones((1, B), jnp.float32),
        (((0,), (0,)), ((), ())),
        preferred_element_type=jnp.float32,
    )


@jax.jit
def _mlp(pooled, w1, b1, w2t, b2r):
    return pl.pallas_call(
        _mlp_body,
        grid=(NVB,),
        in_specs=[
            pl.BlockSpec((B, EMB), lambda j: (0, 0)),
            pl.BlockSpec((EMB, HID), lambda j: (0, 0)),
            pl.BlockSpec((1, HID), lambda j: (0, 0)),
            pl.BlockSpec((BV, HID), lambda j: (j, 0)),
            pl.BlockSpec((1, BV), lambda j: (0, j)),
        ],
        out_specs=pl.BlockSpec((BV, B), lambda j: (j, 0)),
        out_shape=jax.ShapeDtypeStruct((VOCAB, B), jnp.float32),
        scratch_shapes=[pltpu.VMEM((B, HID), jnp.float32)],
        compiler_params=pltpu.CompilerParams(
            dimension_semantics=("arbitrary",),
            vmem_limit_bytes=60 * 1024 * 1024,
        ),
    )(pooled, w1, b1, w2t, b2r)


def kernel(x, emb, W1, b1, W2, b2):
    x_grouped = x.reshape(NW, NCHUNK, IDX_PER_CHUNK).astype(jnp.int32)
    pooled = _pool(x_grouped, emb)
    logits_t = _mlp(pooled, W1, b1.reshape(1, HID), W2.T,
                    b2.reshape(1, VOCAB))
    return logits_t.T
